# baseline (speedup 1.0000x reference)
import jax
import jax.numpy as jnp
from jax.experimental import pallas as pl
from jax.experimental.pallas import tpu as pltpu


def _multidense_kernel(x_ref, w_ref, b_ref, o_ref):
    x = x_ref[0]
    w = w_ref[0]
    b = b_ref[0]
    acc = jnp.dot(x, w, preferred_element_type=jnp.float32)
    o_ref[0] = acc + b


def kernel(inputs, w, b):
    T, B, D_IN = inputs.shape
    D_OUT = w.shape[2]
    NM = 2
    BM = B // NM
    b3 = b.reshape(T, 1, D_OUT)
    return pl.pallas_call(
        _multidense_kernel,
        grid=(T, NM),
        in_specs=[
            pl.BlockSpec((1, BM, D_IN), lambda t, m: (t, m, 0)),
            pl.BlockSpec((1, D_IN, D_OUT), lambda t, m: (t, 0, 0)),
            pl.BlockSpec((1, 1, D_OUT), lambda t, m: (t, 0, 0)),
        ],
        out_specs=pl.BlockSpec((1, BM, D_OUT), lambda t, m: (t, m, 0)),
        out_shape=jax.ShapeDtypeStruct((T, B, D_OUT), jnp.float32),
        compiler_params=pltpu.CompilerParams(
            dimension_semantics=("arbitrary", "arbitrary"),
        ),
    )(inputs, w, b3)


# speedup vs baseline: 1.3609x; 1.3609x over previous
import jax
import jax.numpy as jnp
from jax.experimental import pallas as pl
from jax.experimental.pallas import tpu as pltpu


def _multidense_kernel(x_ref, w_ref, b_ref, o_ref):
    x = x_ref[0]
    w = w_ref[0]
    b = b_ref[0]
    acc = jnp.dot(x, w, preferred_element_type=jnp.float32)
    o_ref[0] = acc + b


def kernel(inputs, w, b):
    T, B, D_IN = inputs.shape
    D_OUT = w.shape[2]
    NM = 1
    BM = B // NM
    b3 = b.reshape(T, 1, D_OUT)
    return pl.pallas_call(
        _multidense_kernel,
        grid=(T, NM),
        in_specs=[
            pl.BlockSpec((1, BM, D_IN), lambda t, m: (t, m, 0)),
            pl.BlockSpec((1, D_IN, D_OUT), lambda t, m: (t, 0, 0)),
            pl.BlockSpec((1, 1, D_OUT), lambda t, m: (t, 0, 0)),
        ],
        out_specs=pl.BlockSpec((1, BM, D_OUT), lambda t, m: (t, m, 0)),
        out_shape=jax.ShapeDtypeStruct((T, B, D_OUT), jnp.float32),
        compiler_params=pltpu.CompilerParams(
            dimension_semantics=("parallel", "arbitrary"),
        ),
    )(inputs, w, b3)


# speedup vs baseline: 1.4297x; 1.0506x over previous
import jax
import jax.numpy as jnp
from jax.experimental import pallas as pl
from jax.experimental.pallas import tpu as pltpu

_TPB = 2


def _multidense_kernel(x_ref, w_ref, b_ref, o_ref):
    for i in range(_TPB):
        acc = jnp.dot(x_ref[i], w_ref[i], preferred_element_type=jnp.float32)
        o_ref[i] = acc + b_ref[i]


def kernel(inputs, w, b):
    T, B, D_IN = inputs.shape
    D_OUT = w.shape[2]
    b3 = b.reshape(T, 1, D_OUT)
    return pl.pallas_call(
        _multidense_kernel,
        grid=(T // _TPB,),
        in_specs=[
            pl.BlockSpec((_TPB, B, D_IN), lambda t: (t, 0, 0)),
            pl.BlockSpec((_TPB, D_IN, D_OUT), lambda t: (t, 0, 0)),
            pl.BlockSpec((_TPB, 1, D_OUT), lambda t: (t, 0, 0)),
        ],
        out_specs=pl.BlockSpec((_TPB, B, D_OUT), lambda t: (t, 0, 0)),
        out_shape=jax.ShapeDtypeStruct((T, B, D_OUT), jnp.float32),
        compiler_params=pltpu.CompilerParams(
            dimension_semantics=("arbitrary",),
        ),
    )(inputs, w, b3)
